# baseline (speedup 1.0000x reference)
import functools

import jax
import jax.numpy as jnp
from jax import lax
from jax.experimental import pallas as pl
from jax.experimental.pallas import tpu as pltpu
from jax.experimental.pallas import tpu_sc as plsc


@functools.lru_cache(maxsize=None)
def _make_lookup(S, T, D, sblk):
    info = plsc.get_sparse_core_info()
    nc, ns, L = info.num_cores, info.num_subcores, info.num_lanes
    nw = nc * ns
    n_sblk = S // sblk
    n_units = T * n_sblk
    assert S % sblk == 0 and n_units % nw == 0
    u_per_w = n_units // nw
    assert u_per_w % 2 == 0 and u_per_w >= 4
    n_groups = sblk // L
    ntc = sblk // 128
    ntr = D // 8
    D2 = 2 * D
    mesh = plsc.VectorSubcoreMesh(core_axis_name="c", subcore_axis_name="s")

    @functools.partial(
        pl.kernel,
        mesh=mesh,
        out_type=jax.ShapeDtypeStruct((T, ntr, S // 128, 8, 128), jnp.float32),
        scratch_types=[
            pltpu.VMEM((u_per_w * sblk,), jnp.int32),
            pltpu.VMEM((2, sblk), jnp.int32),
            pltpu.VMEM((2, sblk, D2), jnp.float32),
            pltpu.VMEM((2, ntr, ntc, 8, 128), jnp.float32),
            pltpu.SemaphoreType.DMA,
            pltpu.SemaphoreType.DMA,
            pltpu.SemaphoreType.DMA,
            pltpu.SemaphoreType.DMA,
        ],
        compiler_params=pltpu.CompilerParams(
            use_tc_tiling_on_sc=False, needs_layout_passes=False),
    )
    def k(ids_hbm, table_hbm, out_hbm, ids_v, idx2_v, g_v, t_v, g0, g1, s0, s1):
        gsem = (g0, g1)
        ssem = (s0, s1)
        wid = lax.axis_index("s") * nc + lax.axis_index("c")
        u0 = wid * u_per_w

        pltpu.sync_copy(ids_hbm.at[pl.ds(u0 * sblk, u_per_w * sblk)], ids_v)

        def prep_gather(lu, b):
            off = lu * sblk
            for kg in range(n_groups):
                v = ids_v[pl.ds(off + kg * L, L)]
                idx2_v[b, pl.ds(kg * L, L)] = lax.shift_right_logical(v, 1)
            pltpu.async_copy(table_hbm.at[idx2_v.at[b]], g_v.at[b], gsem[b])

        def wait_gather(b):
            pltpu.make_async_copy(
                table_hbm.at[idx2_v.at[b]], g_v.at[b], gsem[b]).wait()

        def out_slice(lu):
            u = u0 + lu
            t = u // n_sblk
            sb = u - t * n_sblk
            return out_hbm.at[t, :, pl.ds(sb * ntc, ntc)]

        def start_store(lu, b):
            pltpu.async_copy(t_v.at[b], out_slice(lu), ssem[b])

        def wait_store(lu, b):
            pltpu.make_async_copy(t_v.at[b], out_slice(lu), ssem[b]).wait()

        iota = lax.iota(jnp.int32, L)
        hmod8 = [(jnp.int32(d) + iota) & jnp.int32(7) for d in range(8)]
        cvecs = [jnp.int32(kg * L) + iota for kg in range(8)]

        def select_transpose(lu, b):
            off = lu * sblk
            bvec = jnp.full((L,), b, jnp.int32)
            for tcj in range(ntc):
                tcvec = jnp.full((L,), tcj, jnp.int32)
                pv64 = []
                for kg in range(8):
                    g = tcj * 8 + kg
                    idv = ids_v[pl.ds(off + g * L, L)]
                    pv64.append(
                        lax.shift_left(idv & jnp.int32(1), jnp.int32(6)))

                def trbody(tr4, carry):
                    for half in range(4):
                        tr = tr4 * 4 + half
                        hb = tr * 8
                        trvec = lax.broadcast_in_dim(tr, (L,), ())
                        for kg in range(8):
                            colbase = pv64[kg] + hb
                            rowv = (cvecs[kg] if tcj == 0
                                    else cvecs[kg] + jnp.int32(tcj * 128))
                            for d in range(8):
                                vals = plsc.load_gather(
                                    g_v.at[b], [rowv, colbase + hmod8[d]])
                                plsc.store_scatter(
                                    t_v,
                                    [bvec, trvec, tcvec, hmod8[d], cvecs[kg]],
                                    vals)
                    return carry

                lax.fori_loop(0, ntr // 4, trbody, jnp.int32(0))

        n_pairs = u_per_w // 2
        prep_gather(0, 0)
        prep_gather(1, 1)

        def body(g, carry):
            lu = 2 * g

            def unit(b):
                wait_gather(b)
                pl.when(g > 0)(lambda: wait_store(lu + b - 2, b))
                select_transpose(lu + b, b)
                start_store(lu + b, b)
                pl.when(g < n_pairs - 1)(lambda: prep_gather(lu + b + 2, b))

            unit(0)
            unit(1)
            return carry

        lax.fori_loop(0, n_pairs, body, jnp.int32(0))
        wait_store(u_per_w - 2, 0)
        wait_store(u_per_w - 1, 1)

    return k


def kernel(input_ids, table):
    S, T = input_ids.shape
    D = table.shape[1]
    ids_flat = input_ids.T.reshape(-1).astype(jnp.int32)
    table128 = table.reshape(-1, 2 * D)
    out4 = _make_lookup(S, T, D, 128)(ids_flat, table128)
    return jnp.transpose(out4, (2, 4, 0, 1, 3)).reshape(S, T, D)


# speedup vs baseline: 1.0884x; 1.0884x over previous
import functools

import jax
import jax.numpy as jnp
from jax import lax
from jax.experimental import pallas as pl
from jax.experimental.pallas import tpu as pltpu
from jax.experimental.pallas import tpu_sc as plsc


@functools.lru_cache(maxsize=None)
def _make_lookup(S, T, D, sblk):
    info = plsc.get_sparse_core_info()
    nc, ns, L = info.num_cores, info.num_subcores, info.num_lanes
    nw = nc * ns
    n_sblk = S // sblk
    n_units = T * n_sblk
    assert S % sblk == 0 and n_units % nw == 0
    u_per_w = n_units // nw
    assert u_per_w % 2 == 0 and u_per_w >= 4
    n_groups = sblk // L
    ntc = sblk // 128
    ntr = D // 8
    D2 = 2 * D
    mesh = plsc.VectorSubcoreMesh(core_axis_name="c", subcore_axis_name="s")

    @functools.partial(
        pl.kernel,
        mesh=mesh,
        out_type=jax.ShapeDtypeStruct((T, ntr, S // 128, 8, 128), jnp.float32),
        scratch_types=[
            pltpu.VMEM((u_per_w * sblk,), jnp.int32),
            pltpu.VMEM((2, sblk), jnp.int32),
            pltpu.VMEM((2, sblk, D2), jnp.float32),
            pltpu.VMEM((2, ntr, ntc, 8, 128), jnp.float32),
            pltpu.SemaphoreType.DMA,
            pltpu.SemaphoreType.DMA,
            pltpu.SemaphoreType.DMA,
            pltpu.SemaphoreType.DMA,
        ],
        compiler_params=pltpu.CompilerParams(
            use_tc_tiling_on_sc=False, needs_layout_passes=False),
    )
    def k(ids_hbm, table_hbm, out_hbm, ids_v, idx2_v, g_v, t_v, g0, g1, s0, s1):
        gsem = (g0, g1)
        ssem = (s0, s1)
        wid = lax.axis_index("s") * nc + lax.axis_index("c")
        u0 = wid * u_per_w

        pltpu.sync_copy(ids_hbm.at[pl.ds(u0 * sblk, u_per_w * sblk)], ids_v)

        def prep_gather(lu, b):
            off = lu * sblk
            for kg in range(n_groups):
                v = ids_v[pl.ds(off + kg * L, L)]
                idx2_v[b, pl.ds(kg * L, L)] = lax.shift_right_logical(v, 1)
            pltpu.async_copy(table_hbm.at[idx2_v.at[b]], g_v.at[b], gsem[b])

        def wait_gather(b):
            pltpu.make_async_copy(
                table_hbm.at[idx2_v.at[b]], g_v.at[b], gsem[b]).wait()

        def out_slice(lu):
            u = u0 + lu
            t = u // n_sblk
            sb = u - t * n_sblk
            return out_hbm.at[t, :, pl.ds(sb * ntc, ntc)]

        def start_store(lu, b):
            pltpu.async_copy(t_v.at[b], out_slice(lu), ssem[b])

        def wait_store(lu, b):
            pltpu.make_async_copy(t_v.at[b], out_slice(lu), ssem[b]).wait()

        iota = lax.iota(jnp.int32, L)
        hmod8 = [(jnp.int32(d) + iota) & jnp.int32(7) for d in range(8)]
        cvecs = [jnp.int32(kg * L) + iota for kg in range(8)]

        def select_transpose(lu, b):
            off = lu * sblk
            bvec = jnp.full((L,), b, jnp.int32)
            for tcj in range(ntc):
                tcvec = jnp.full((L,), tcj, jnp.int32)
                pv64 = []
                for kg in range(8):
                    g = tcj * 8 + kg
                    idv = ids_v[pl.ds(off + g * L, L)]
                    pv64.append(
                        lax.shift_left(idv & jnp.int32(1), jnp.int32(6)))

                def trbody(tr4, carry):
                    for half in range(2):
                        tr = tr4 * 2 + half
                        hb = tr * 8
                        trvec = lax.broadcast_in_dim(tr, (L,), ())
                        for kg in range(8):
                            colbase = pv64[kg] + hb
                            rowv = (cvecs[kg] if tcj == 0
                                    else cvecs[kg] + jnp.int32(tcj * 128))
                            for d in range(8):
                                vals = plsc.load_gather(
                                    g_v.at[b], [rowv, colbase + hmod8[d]])
                                plsc.store_scatter(
                                    t_v,
                                    [bvec, trvec, tcvec, hmod8[d], cvecs[kg]],
                                    vals)
                    return carry

                lax.fori_loop(0, ntr // 2, trbody, jnp.int32(0))

        n_pairs = u_per_w // 2
        prep_gather(0, 0)
        prep_gather(1, 1)

        def body(g, carry):
            lu = 2 * g

            def unit(b):
                wait_gather(b)
                pl.when(g > 0)(lambda: wait_store(lu + b - 2, b))
                select_transpose(lu + b, b)
                start_store(lu + b, b)
                pl.when(g < n_pairs - 1)(lambda: prep_gather(lu + b + 2, b))

            unit(0)
            unit(1)
            return carry

        lax.fori_loop(0, n_pairs, body, jnp.int32(0))
        wait_store(u_per_w - 2, 0)
        wait_store(u_per_w - 1, 1)

    return k


def kernel(input_ids, table):
    S, T = input_ids.shape
    D = table.shape[1]
    ids_flat = input_ids.T.reshape(-1).astype(jnp.int32)
    table128 = table.reshape(-1, 2 * D)
    out4 = _make_lookup(S, T, D, 128)(ids_flat, table128)
    return jnp.transpose(out4, (2, 4, 0, 1, 3)).reshape(S, T, D)
